# baseline (speedup 1.0000x reference)
import functools

import jax
import jax.numpy as jnp
from jax import lax
from jax.experimental import pallas as pl
from jax.experimental.pallas import tpu as pltpu
from jax.experimental.pallas import tpu_sc as plsc

DIM = 256
K = 1024
N = 16 * 576
TN = 512

NC = 2
NS = 16
NW = NC * NS
B_PER_W = N // NW
CHUNK = 96
N_CHUNK = B_PER_W // CHUNK


def _argmin_body(x_ref, e_ref, idx_ref):
    x = x_ref[...]
    e = e_ref[...]
    s = lax.dot_general(x, e, (((1,), (1,)), ((), ())),
                        preferred_element_type=jnp.float32)
    xnorm = jnp.sum(x * x, axis=1, keepdims=True)
    enorm = jnp.sum(e * e, axis=1)[None, :]
    dist = -(xnorm - 2.0 * s + enorm)
    iota = lax.broadcasted_iota(jnp.int32, (TN, K), 1)
    m = jnp.max(dist, axis=1, keepdims=True)
    idx_ref[...] = jnp.min(jnp.where(dist == m, iota, K), axis=1)


def _sc_gather_body(e_hbm, idx_hbm, out_hbm, idx_v, rows_v, sem):
    wid = lax.axis_index("s") * NC + lax.axis_index("c")
    base = wid * B_PER_W
    pltpu.sync_copy(idx_hbm.at[pl.ds(base, B_PER_W)], idx_v)
    copies = []
    for c in range(N_CHUNK):
        copies.append(pltpu.async_copy(
            e_hbm.at[idx_v.at[pl.ds(c * CHUNK, CHUNK)]],
            rows_v.at[pl.ds(c * CHUNK, CHUNK)], sem))
    for cp in copies:
        cp.wait()
    pltpu.sync_copy(rows_v, out_hbm.at[pl.ds(base, B_PER_W)])


@jax.jit
def _vq(flat, embed):
    idx = pl.pallas_call(
        _argmin_body,
        grid=(N // TN,),
        in_specs=[
            pl.BlockSpec((TN, DIM), lambda i: (i, 0)),
            pl.BlockSpec((K, DIM), lambda i: (0, 0)),
        ],
        out_specs=pl.BlockSpec((TN,), lambda i: (i,)),
        out_shape=jax.ShapeDtypeStruct((N,), jnp.int32),
    )(flat, embed)

    gather = pl.kernel(
        _sc_gather_body,
        mesh=plsc.VectorSubcoreMesh(core_axis_name="c", subcore_axis_name="s"),
        out_type=jax.ShapeDtypeStruct((N, DIM), jnp.float32),
        scratch_types=[
            pltpu.VMEM((B_PER_W,), jnp.int32),
            pltpu.VMEM((B_PER_W, DIM), jnp.float32),
            pltpu.SemaphoreType.DMA,
        ],
    )
    quantize = gather(embed, idx)
    return quantize, idx


def kernel(x, embed):
    shape = x.shape
    flat = x.reshape(-1, shape[-1])
    quantize, idx = _vq(flat, embed)
    return quantize.reshape(shape), idx.reshape(shape[:-1])


# speedup vs baseline: 1.2645x; 1.2645x over previous
import functools

import jax
import jax.numpy as jnp
from jax import lax
from jax.experimental import pallas as pl
from jax.experimental.pallas import tpu as pltpu
from jax.experimental.pallas import tpu_sc as plsc

DIM = 256
K = 1024
N = 16 * 576
TN = 512

NC = 2
NS = 16
NW = NC * NS
B_PER_W = N // NW
CHUNK = 96
N_CHUNK = B_PER_W // CHUNK


def _argmin_body(x_ref, e_ref, idx_ref):
    x = x_ref[...]
    e = e_ref[...]
    s2 = lax.dot_general(e, x * (-2.0), (((1,), (1,)), ((), ())),
                         preferred_element_type=jnp.float32)
    xnorm = jnp.transpose(jnp.sum(x * x, axis=1, keepdims=True))
    enorm = jnp.sum(e * e, axis=1)[:, None]
    u = (xnorm + s2) + enorm
    del u
    u = (xnorm + s2) + enorm
    m = jnp.min(u, axis=0)[None, :]
    iota = lax.broadcasted_iota(jnp.int32, (K, 1), 0).astype(jnp.float32)
    penal = jnp.where(u == m, iota, float(K))
    idx_ref[...] = jnp.min(penal, axis=0).astype(jnp.int32)


def _sc_gather_body(e_hbm, idx_hbm, out_hbm, idx_v, rows_v, sem):
    wid = lax.axis_index("s") * NC + lax.axis_index("c")
    base = wid * B_PER_W
    pltpu.sync_copy(idx_hbm.at[pl.ds(base, B_PER_W)], idx_v)
    copies = []
    for c in range(N_CHUNK):
        copies.append(pltpu.async_copy(
            e_hbm.at[idx_v.at[pl.ds(c * CHUNK, CHUNK)]],
            rows_v.at[pl.ds(c * CHUNK, CHUNK)], sem))
    for cp in copies:
        cp.wait()
    pltpu.sync_copy(rows_v, out_hbm.at[pl.ds(base, B_PER_W)])


@jax.jit
def _vq(flat, embed):
    idx = pl.pallas_call(
        _argmin_body,
        grid=(N // TN,),
        in_specs=[
            pl.BlockSpec((TN, DIM), lambda i: (i, 0)),
            pl.BlockSpec((K, DIM), lambda i: (0, 0)),
        ],
        out_specs=pl.BlockSpec((TN,), lambda i: (i,)),
        out_shape=jax.ShapeDtypeStruct((N,), jnp.int32),
    )(flat, embed)

    gather = pl.kernel(
        _sc_gather_body,
        mesh=plsc.VectorSubcoreMesh(core_axis_name="c", subcore_axis_name="s"),
        out_type=jax.ShapeDtypeStruct((N, DIM), jnp.float32),
        scratch_types=[
            pltpu.VMEM((B_PER_W,), jnp.int32),
            pltpu.VMEM((B_PER_W, DIM), jnp.float32),
            pltpu.SemaphoreType.DMA,
        ],
    )
    quantize = gather(embed, idx)
    return quantize, idx


def kernel(x, embed):
    shape = x.shape
    flat = x.reshape(-1, shape[-1])
    quantize, idx = _vq(flat, embed)
    return quantize.reshape(shape), idx.reshape(shape[:-1])


# speedup vs baseline: 1.3931x; 1.1017x over previous
import functools

import jax
import jax.numpy as jnp
from jax import lax
from jax.experimental import pallas as pl
from jax.experimental.pallas import tpu as pltpu
from jax.experimental.pallas import tpu_sc as plsc

DIM = 256
K = 1024
N = 16 * 576
TN = 1024

NC = 2
NS = 16
NW = NC * NS
B_PER_W = N // NW
CHUNK = 96
N_CHUNK = B_PER_W // CHUNK


def _argmin_body(x_ref, e_ref, idx_ref):
    x = x_ref[...]
    e = e_ref[...]
    s2 = lax.dot_general(e, x * (-2.0), (((1,), (1,)), ((), ())),
                         preferred_element_type=jnp.float32)
    xnorm = jnp.transpose(jnp.sum(x * x, axis=1, keepdims=True))
    enorm = jnp.sum(e * e, axis=1)[:, None]
    u = (xnorm + s2) + enorm
    del u
    u = (xnorm + s2) + enorm
    m = jnp.min(u, axis=0)[None, :]
    iota = lax.broadcasted_iota(jnp.int32, (K, 1), 0).astype(jnp.float32)
    penal = jnp.where(u == m, iota, float(K))
    idx_ref[...] = jnp.min(penal, axis=0).astype(jnp.int32)


def _sc_gather_body(e_hbm, idx_hbm, out_hbm, idx_v, rows_v, sem):
    wid = lax.axis_index("s") * NC + lax.axis_index("c")
    base = wid * B_PER_W
    pltpu.sync_copy(idx_hbm.at[pl.ds(base, B_PER_W)], idx_v)
    copies = []
    for c in range(N_CHUNK):
        copies.append(pltpu.async_copy(
            e_hbm.at[idx_v.at[pl.ds(c * CHUNK, CHUNK)]],
            rows_v.at[pl.ds(c * CHUNK, CHUNK)], sem))
    for cp in copies:
        cp.wait()
    pltpu.sync_copy(rows_v, out_hbm.at[pl.ds(base, B_PER_W)])


@jax.jit
def _vq(flat, embed):
    idx = pl.pallas_call(
        _argmin_body,
        grid=(N // TN,),
        in_specs=[
            pl.BlockSpec((TN, DIM), lambda i: (i, 0)),
            pl.BlockSpec((K, DIM), lambda i: (0, 0)),
        ],
        out_specs=pl.BlockSpec((TN,), lambda i: (i,)),
        out_shape=jax.ShapeDtypeStruct((N,), jnp.int32),
    )(flat, embed)

    gather = pl.kernel(
        _sc_gather_body,
        mesh=plsc.VectorSubcoreMesh(core_axis_name="c", subcore_axis_name="s"),
        out_type=jax.ShapeDtypeStruct((N, DIM), jnp.float32),
        scratch_types=[
            pltpu.VMEM((B_PER_W,), jnp.int32),
            pltpu.VMEM((B_PER_W, DIM), jnp.float32),
            pltpu.SemaphoreType.DMA,
        ],
    )
    quantize = gather(embed, idx)
    return quantize, idx


def kernel(x, embed):
    shape = x.shape
    flat = x.reshape(-1, shape[-1])
    quantize, idx = _vq(flat, embed)
    return quantize.reshape(shape), idx.reshape(shape[:-1])


# speedup vs baseline: 1.9123x; 1.3727x over previous
import jax
import jax.numpy as jnp
from jax import lax
from jax.experimental import pallas as pl

DIM = 256
K = 1024
N = 16 * 576
TN = 1024


def _vq_body(x_ref, e_ref, q_ref, idx_ref):
    x = x_ref[...]
    e = e_ref[...]
    s2 = lax.dot_general(e, x * (-2.0), (((1,), (1,)), ((), ())),
                         preferred_element_type=jnp.float32)
    xnorm = jnp.transpose(jnp.sum(x * x, axis=1, keepdims=True))
    enorm = jnp.sum(e * e, axis=1)[:, None]
    u = (xnorm + s2) + enorm
    m = jnp.min(u, axis=0)[None, :]
    iota = lax.broadcasted_iota(jnp.int32, (K, 1), 0).astype(jnp.float32)
    penal = jnp.where(u == m, iota, float(K))
    idxf = jnp.min(penal, axis=0)
    idx_ref[...] = idxf.astype(jnp.int32)
    onehot = (iota == idxf[None, :]).astype(jnp.float32)
    q_ref[...] = lax.dot_general(onehot, e, (((0,), (0,)), ((), ())),
                                 preferred_element_type=jnp.float32)


@jax.jit
def _vq(flat, embed):
    return pl.pallas_call(
        _vq_body,
        grid=(N // TN,),
        in_specs=[
            pl.BlockSpec((TN, DIM), lambda i: (i, 0)),
            pl.BlockSpec((K, DIM), lambda i: (0, 0)),
        ],
        out_specs=[
            pl.BlockSpec((TN, DIM), lambda i: (i, 0)),
            pl.BlockSpec((TN,), lambda i: (i,)),
        ],
        out_shape=[
            jax.ShapeDtypeStruct((N, DIM), jnp.float32),
            jax.ShapeDtypeStruct((N,), jnp.int32),
        ],
    )(flat, embed)


def kernel(x, embed):
    shape = x.shape
    flat = x.reshape(-1, shape[-1])
    quantize, idx = _vq(flat, embed)
    return quantize.reshape(shape), idx.reshape(shape[:-1])
